# Initial kernel scaffold; baseline (speedup 1.0000x reference)
import jax
import jax.numpy as jnp
from jax.experimental import pallas as pl


def kernel(input_index, targets, token_embedding_table):
    raise NotImplementedError("write your pallas kernel here")


# speedup vs baseline: 1.6155x; 1.6155x over previous
import functools

import jax
import jax.numpy as jnp
from jax import lax
from jax.experimental import pallas as pl
from jax.experimental.pallas import tpu as pltpu
from jax.experimental.pallas import tpu_sc as plsc

V = 1000
N_TOK = 1024 * 50
NC, NS, LANES = 2, 16, 16
NW = NC * NS
TOK_PER_W = N_TOK // NW
CHUNK = 32
N_CHUNK = TOK_PER_W // CHUNK


def _lse_body(tab_ref, lse_ref):
    t = tab_ref[...]
    m = jnp.max(t, axis=1)
    lse_ref[...] = m + jnp.log(jnp.sum(jnp.exp(t - m[:, None]), axis=1))


def _row_lse(table):
    return pl.pallas_call(
        _lse_body,
        out_shape=jax.ShapeDtypeStruct((V,), jnp.float32),
    )(table)


_MESH = plsc.VectorSubcoreMesh(core_axis_name="c", subcore_axis_name="s")


@functools.partial(
    pl.kernel,
    out_type=[
        jax.ShapeDtypeStruct((N_TOK, V), jnp.float32),
        jax.ShapeDtypeStruct((NW, LANES), jnp.float32),
    ],
    mesh=_MESH,
    compiler_params=pltpu.CompilerParams(use_tc_tiling_on_sc=False,
                                         needs_layout_passes=False),
    scratch_types=[
        pltpu.VMEM((TOK_PER_W,), jnp.int32),
        pltpu.VMEM((TOK_PER_W,), jnp.int32),
        pltpu.VMEM((V,), jnp.float32),
        pltpu.VMEM((CHUNK, V), jnp.float32),
        pltpu.VMEM((LANES,), jnp.float32),
        pltpu.SemaphoreType.DMA,
    ],
)
def _sc_main(table_hbm, idx_hbm, tgt_hbm, lse_hbm, x_hbm, part_hbm,
             idx_v, tgt_v, lse_v, rows_v, part_v, sem):
    wid = lax.axis_index("s") * NC + lax.axis_index("c")
    base = wid * TOK_PER_W
    pltpu.sync_copy(idx_hbm.at[pl.ds(base, TOK_PER_W)], idx_v)
    pltpu.sync_copy(tgt_hbm.at[pl.ds(base, TOK_PER_W)], tgt_v)
    pltpu.sync_copy(lse_hbm, lse_v)

    def chunk_body(k, acc):
        off = k * CHUNK
        idx_chunk = idx_v.at[pl.ds(off, CHUNK)]
        pltpu.async_copy(table_hbm.at[idx_chunk], rows_v, sem).wait()
        for j in range(CHUNK // LANES):
            s = pl.ds(off + j * LANES, LANES)
            idxv = idx_v[s]
            tgtv = tgt_v[s]
            rloc = lax.iota(jnp.int32, LANES) + j * LANES
            picked = plsc.load_gather(rows_v, [rloc, tgtv])
            lseg = plsc.load_gather(lse_v, [idxv])
            acc = acc + (lseg - picked)
        pltpu.sync_copy(rows_v, x_hbm.at[pl.ds(base + off, CHUNK)])
        return acc

    acc = lax.fori_loop(0, N_CHUNK, chunk_body,
                        jnp.zeros((LANES,), jnp.float32))
    part_v[...] = acc
    pltpu.sync_copy(part_v, part_hbm.at[wid])


def _loss_body(part_ref, out_ref):
    out_ref[0, 0] = jnp.sum(part_ref[...]) * (1.0 / N_TOK)


def _final_loss(partials):
    return pl.pallas_call(
        _loss_body,
        out_shape=jax.ShapeDtypeStruct((1, 1), jnp.float32),
        out_specs=pl.BlockSpec(memory_space=pltpu.SMEM),
    )(partials)


def kernel(input_index, targets, token_embedding_table):
    idx = input_index.reshape(-1).astype(jnp.int32)
    tgt = targets.reshape(-1).astype(jnp.int32)
    table = token_embedding_table
    lse = _row_lse(table)
    x, partials = _sc_main(table, idx, tgt, lse)
    loss = _final_loss(partials)[0, 0]
    return (x, loss)


# speedup vs baseline: 1.7132x; 1.0604x over previous
import functools

import jax
import jax.numpy as jnp
from jax import lax
from jax.experimental import pallas as pl
from jax.experimental.pallas import tpu as pltpu
from jax.experimental.pallas import tpu_sc as plsc

V = 1000
N_TOK = 1024 * 50
NC, NS, LANES = 2, 16, 16
NW = NC * NS
TOK_PER_W = N_TOK // NW
CHUNK = 16
N_CHUNK = TOK_PER_W // CHUNK
NBUF = 4
LAG = 2
N_GROUP = N_CHUNK // NBUF


def _lse_body(tab_ref, lse_ref):
    t = tab_ref[...]
    m = jnp.max(t, axis=1)
    lse_ref[...] = m + jnp.log(jnp.sum(jnp.exp(t - m[:, None]), axis=1))


def _row_lse(table):
    return pl.pallas_call(
        _lse_body,
        out_shape=jax.ShapeDtypeStruct((V,), jnp.float32),
    )(table)


_MESH = plsc.VectorSubcoreMesh(core_axis_name="c", subcore_axis_name="s")


@functools.partial(
    pl.kernel,
    out_type=[
        jax.ShapeDtypeStruct((N_TOK, V), jnp.float32),
        jax.ShapeDtypeStruct((NW, LANES), jnp.float32),
    ],
    mesh=_MESH,
    compiler_params=pltpu.CompilerParams(use_tc_tiling_on_sc=False,
                                         needs_layout_passes=False),
    scratch_types=[
        pltpu.VMEM((TOK_PER_W,), jnp.int32),
        pltpu.VMEM((TOK_PER_W,), jnp.int32),
        pltpu.VMEM((V,), jnp.float32),
        [pltpu.VMEM((CHUNK, V), jnp.float32)] * NBUF,
        pltpu.VMEM((LANES,), jnp.float32),
        [pltpu.SemaphoreType.DMA] * NBUF,
        [pltpu.SemaphoreType.DMA] * NBUF,
    ],
)
def _sc_main(table_hbm, idx_hbm, tgt_hbm, lse_hbm, x_hbm, part_hbm,
             idx_v, tgt_v, lse_v, rows, part_v, sem_g, sem_s):
    wid = lax.axis_index("s") * NC + lax.axis_index("c")
    base = wid * TOK_PER_W
    pltpu.sync_copy(idx_hbm.at[pl.ds(base, TOK_PER_W)], idx_v)
    pltpu.sync_copy(tgt_hbm.at[pl.ds(base, TOK_PER_W)], tgt_v)
    pltpu.sync_copy(lse_hbm, lse_v)

    def start_gather(k, b):
        idx_chunk = idx_v.at[pl.ds(k * CHUNK, CHUNK)]
        pltpu.make_async_copy(table_hbm.at[idx_chunk], rows[b],
                              sem_g[b]).start()

    def wait_gather(b):
        pltpu.make_async_copy(table_hbm.at[idx_v.at[pl.ds(0, CHUNK)]],
                              rows[b], sem_g[b]).wait()

    def start_scatter(k, b):
        pltpu.make_async_copy(rows[b], x_hbm.at[pl.ds(base + k * CHUNK,
                                                      CHUNK)],
                              sem_s[b]).start()

    def wait_scatter(b):
        pltpu.make_async_copy(rows[b], x_hbm.at[pl.ds(base, CHUNK)],
                              sem_s[b]).wait()

    for j in range(NBUF):
        start_gather(j, j)

    rloc = lax.iota(jnp.int32, LANES)

    def group_body(g, acc):
        for b in range(NBUF):
            k = g * NBUF + b
            wait_gather(b)
            s = pl.ds(k * CHUNK, LANES)
            picked = plsc.load_gather(rows[b], [rloc, tgt_v[s]])
            lseg = plsc.load_gather(lse_v, [idx_v[s]])
            acc = acc + (lseg - picked)
            start_scatter(k, b)
            b2 = (b - LAG) % NBUF

            @pl.when(jnp.logical_and(k >= LAG, k < N_CHUNK - NBUF + LAG))
            def _():
                wait_scatter(b2)
                start_gather(k - LAG + NBUF, b2)
        return acc

    acc = lax.fori_loop(0, N_GROUP, group_body,
                        jnp.zeros((LANES,), jnp.float32))
    for b in range(NBUF):
        wait_scatter(b)
    part_v[...] = acc
    pltpu.sync_copy(part_v, part_hbm.at[wid])


def _loss_body(part_ref, out_ref):
    out_ref[0, 0] = jnp.sum(part_ref[...]) * (1.0 / N_TOK)


def _final_loss(partials):
    return pl.pallas_call(
        _loss_body,
        out_shape=jax.ShapeDtypeStruct((1, 1), jnp.float32),
        out_specs=pl.BlockSpec(memory_space=pltpu.SMEM),
    )(partials)


def kernel(input_index, targets, token_embedding_table):
    idx = input_index.reshape(-1).astype(jnp.int32)
    tgt = targets.reshape(-1).astype(jnp.int32)
    table = token_embedding_table
    lse = _row_lse(table)
    x, partials = _sc_main(table, idx, tgt, lse)
    loss = _final_loss(partials)[0, 0]
    return (x, loss)
